# baseline (speedup 1.0000x reference)
import functools

import jax
import jax.numpy as jnp
from jax import lax
from jax.experimental import pallas as pl
from jax.experimental.pallas import tpu as pltpu
from jax.experimental.pallas import tpu_sc as plsc

N = 10000
E = 320000
D = 128
DOUT = 64
EPS = 1e-5

NC = 2
NW = 32
NPT = 313
NPAD = NW * NPT
FMAX = 3.402823466e38

SCH = 3200
NSCAN = E // SCH
FLUSH = 1024
OBUF = 5120
OCAP = E + 4096

GCH = 128

_mesh = plsc.VectorSubcoreMesh(core_axis_name="c", subcore_axis_name="s")
_params = pltpu.CompilerParams(needs_layout_passes=False)


def _worker_id():
    return lax.axis_index("s") * NC + lax.axis_index("c")


@functools.partial(
    pl.kernel,
    out_type=(
        jax.ShapeDtypeStruct((NW * OCAP,), jnp.int32),
        jax.ShapeDtypeStruct((NW * OCAP,), jnp.int32),
        jax.ShapeDtypeStruct((NW * 16,), jnp.int32),
    ),
    mesh=_mesh,
    compiler_params=_params,
    scratch_types=[
        pltpu.VMEM((SCH,), jnp.int32),
        pltpu.VMEM((SCH,), jnp.int32),
        pltpu.VMEM((SCH,), jnp.int32),
        pltpu.VMEM((SCH,), jnp.int32),
        pltpu.VMEM((OBUF,), jnp.int32),
        pltpu.VMEM((OBUF,), jnp.int32),
        pltpu.VMEM((16,), jnp.int32),
        pltpu.SemaphoreType.DMA,
        pltpu.SemaphoreType.DMA,
    ],
)
def _compact(src_hbm, dst_hbm, esrc_hbm, edst_hbm, cnt_hbm,
             sbufA, dbufA, sbufB, dbufB, osrc, odst, cbuf, semA, semB):
    w = _worker_id()
    lo = w * NPT
    hi = lo + NPT

    def start_stage(ci, sb, db, sem):
        ci = jnp.minimum(ci, NSCAN - 1)
        so = pl.multiple_of(ci * SCH, 8)
        pltpu.async_copy(src_hbm.at[pl.ds(so, SCH)], sb, sem)
        pltpu.async_copy(dst_hbm.at[pl.ds(so, SCH)], db, sem)

    def wait_stage(sb, db, sem):
        pltpu.make_async_copy(src_hbm.at[pl.ds(0, SCH)], sb, sem).wait()
        pltpu.make_async_copy(dst_hbm.at[pl.ds(0, SCH)], db, sem).wait()

    def process(sb, db, carry):
        cur, base = carry

        def quad(q, cur2):
            tot = cur2
            for k in range(4):
                off = q * 64 + k * 16
                dv = db[pl.ds(off, 16)]
                sv = sb[pl.ds(off, 16)]
                m = (dv >= lo) & (dv < hi)
                cs = plsc.cumsum(m.astype(jnp.int32))
                pos = tot + cs - 1
                plsc.store_scatter(osrc, [pos], sv, mask=m)
                plsc.store_scatter(odst, [pos], dv - lo, mask=m)
                tot = tot + cs[15]
            return tot

        cur = lax.fori_loop(0, SCH // 64, quad, cur)

        nfl = cur // FLUSH
        for k in range(4):
            @pl.when(nfl >= k + 1)
            def _(k=k):
                fo = pl.multiple_of(w * OCAP + base + k * FLUSH, 8)
                pltpu.sync_copy(osrc.at[pl.ds(k * FLUSH, FLUSH)],
                                esrc_hbm.at[pl.ds(fo, FLUSH)])
                pltpu.sync_copy(odst.at[pl.ds(k * FLUSH, FLUSH)],
                                edst_hbm.at[pl.ds(fo, FLUSH)])

        sh = nfl * FLUSH

        @pl.when(nfl >= 1)
        def _():
            def mv(j, _):
                osrc[pl.ds(j * 16, 16)] = osrc[pl.ds(sh + j * 16, 16)]
                odst[pl.ds(j * 16, 16)] = odst[pl.ds(sh + j * 16, 16)]
                return 0
            lax.fori_loop(0, FLUSH // 16, mv, 0)

        return cur - sh, base + sh

    start_stage(0, sbufA, dbufA, semA)

    def pair(pi, carry):
        c0 = pi * 2
        start_stage(c0 + 1, sbufB, dbufB, semB)
        wait_stage(sbufA, dbufA, semA)
        carry = process(sbufA, dbufA, carry)
        start_stage(c0 + 2, sbufA, dbufA, semA)
        wait_stage(sbufB, dbufB, semB)
        return process(sbufB, dbufB, carry)

    cur, base = lax.fori_loop(0, NSCAN // 2, pair,
                              (jnp.int32(0), jnp.int32(0)))
    wait_stage(sbufA, dbufA, semA)

    fo = pl.multiple_of(w * OCAP + base, 8)
    pltpu.sync_copy(osrc.at[pl.ds(0, FLUSH)], esrc_hbm.at[pl.ds(fo, FLUSH)])
    pltpu.sync_copy(odst.at[pl.ds(0, FLUSH)], edst_hbm.at[pl.ds(fo, FLUSH)])
    cbuf[...] = jnp.zeros((16,), jnp.int32) + (base + cur)
    pltpu.sync_copy(cbuf, cnt_hbm.at[pl.ds(pl.multiple_of(w * 16, 8), 16)])


SUP = 12288
BCH = 512

@functools.partial(
    pl.kernel,
    out_type=jax.ShapeDtypeStruct((NW * NPT * D,), jnp.float32),
    mesh=_mesh,
    compiler_params=_params,
    scratch_types=[
        pltpu.VMEM(((NPT + 1) * D,), jnp.float32),
        pltpu.VMEM((SUP,), jnp.int32),
        pltpu.VMEM((SUP,), jnp.int32),
        pltpu.VMEM((BCH, D), jnp.float32),
        pltpu.VMEM((16,), jnp.int32),
        pltpu.SemaphoreType.DMA,
    ],
)
def _segmin(x_hbm, esrc_hbm, edst_hbm, cnt_hbm, agg_hbm,
            acc, idxs, dsts, rows, cntb, sem):
    w = _worker_id()

    fv = jnp.full((16,), FMAX, jnp.float32)

    def init(i, _):
        acc[pl.ds(i * 16, 16)] = fv
        return 0
    lax.fori_loop(0, (NPT + 1) * D // 16, init, 0)

    pltpu.sync_copy(cnt_hbm.at[pl.ds(pl.multiple_of(w * 16, 8), 16)], cntb)
    cnt = cntb[...][0]
    iota = lax.broadcasted_iota(jnp.int32, (16,), 0)

    def sup(si, _):
        so = pl.multiple_of(w * OCAP + si * SUP, 8)
        pltpu.sync_copy(esrc_hbm.at[pl.ds(so, SUP)], idxs)
        pltpu.sync_copy(edst_hbm.at[pl.ds(so, SUP)], dsts)
        rem = jnp.minimum(cnt - si * SUP, SUP)
        nb = (rem + BCH - 1) // BCH

        def batch(bi, _):
            b0 = pl.multiple_of(bi * BCH, 8)
            for g in range(BCH // 16):
                ev = si * SUP + b0 + g * 16 + iota
                valid = ev < cnt
                iv = idxs[pl.ds(b0 + g * 16, 16)]
                dv = dsts[pl.ds(b0 + g * 16, 16)]
                idxs[pl.ds(b0 + g * 16, 16)] = jnp.where(valid, iv, 0)
                dsts[pl.ds(b0 + g * 16, 16)] = jnp.where(valid, dv, NPT)
            for k in range(BCH // 128):
                pltpu.async_copy(
                    x_hbm.at[idxs.at[pl.ds(b0 + k * 128, 128)]],
                    rows.at[pl.ds(k * 128, 128), :], sem)
            for k in range(BCH // 128):
                pltpu.make_async_copy(
                    x_hbm.at[idxs.at[pl.ds(b0 + k * 128, 128)]],
                    rows.at[pl.ds(k * 128, 128), :], sem).wait()

            def egroup(g, _):
                dvec = dsts[pl.ds(b0 + g * 16, 16)]
                for l in range(16):
                    d = dvec[l]
                    ab = d * D
                    e = g * 16 + l
                    for j in range(D // 16):
                        av = acc[pl.ds(ab + j * 16, 16)]
                        rv = rows[e, pl.ds(j * 16, 16)]
                        acc[pl.ds(ab + j * 16, 16)] = jnp.minimum(av, rv)
                return 0
            lax.fori_loop(0, BCH // 16, egroup, 0)
            return 0

        lax.fori_loop(0, nb, batch, 0)
        return 0

    lax.fori_loop(0, (cnt + SUP - 1) // SUP, sup, 0)
    pltpu.sync_copy(acc.at[pl.ds(0, NPT * D)],
                    agg_hbm.at[pl.ds(pl.multiple_of(w * NPT * D, 8), NPT * D)])


def _dense_body(agg_ref, h_ref, wl_ref, b_ref, wr_ref, g_ref, be_ref, o_ref):
    a = agg_ref[0:N, :]
    a = jnp.where(a > 3.0e38, 0.0, a)
    z = (jnp.dot(a, wl_ref[...], preferred_element_type=jnp.float32)
         + b_ref[...]
         + jnp.dot(h_ref[...], wr_ref[...], preferred_element_type=jnp.float32))
    m = jnp.mean(z, axis=0, keepdims=True)
    v = jnp.mean((z - m) ** 2, axis=0, keepdims=True)
    zn = (z - m) * lax.rsqrt(v + EPS) * g_ref[...] + be_ref[...]
    o_ref[...] = jnp.maximum(zn, 0.0)


def _dense(agg, h, wl, b, wr, g, be):
    return pl.pallas_call(
        _dense_body,
        out_shape=jax.ShapeDtypeStruct((N, D), jnp.float32),
    )(agg, h, wl, b, wr, g, be)


def _final_body(agg_ref, h_ref, wl_ref, b_ref, wr_ref, o_ref):
    a = agg_ref[0:N, :]
    a = jnp.where(a > 3.0e38, 0.0, a)
    z = (jnp.dot(a, wl_ref[...], preferred_element_type=jnp.float32)
         + b_ref[...]
         + jnp.dot(h_ref[...], wr_ref[...], preferred_element_type=jnp.float32))
    mx = jnp.max(z, axis=1, keepdims=True)
    ez = jnp.exp(z - mx)
    ls = jnp.log(jnp.sum(ez, axis=1, keepdims=True)) + mx
    o_ref[...] = z - ls


def _final(agg, h, wl, b, wr):
    return pl.pallas_call(
        _final_body,
        out_shape=jax.ShapeDtypeStruct((N, DOUT), jnp.float32),
    )(agg, h, wl, b, wr)


def kernel(x, edge_index, W1l, b1, W1r, g1, be1,
           W2l, b2, W2r, g2, be2, W3l, b3, W3r):
    src = edge_index[0]
    dst = edge_index[1]
    esrc, edst, cnts = _compact(src, dst)

    def seg(h):
        return _segmin(h, esrc, edst, cnts).reshape(NPAD, D)

    h1 = _dense(seg(x), x, W1l, b1.reshape(1, D), W1r,
                g1.reshape(1, D), be1.reshape(1, D))
    h2 = _dense(seg(h1), h1, W2l, b2.reshape(1, D), W2r,
                g2.reshape(1, D), be2.reshape(1, D))
    return _final(seg(h2), h2, W3l, b3.reshape(1, DOUT), W3r)


# speedup vs baseline: 1.2354x; 1.2354x over previous
import functools

import jax
import jax.numpy as jnp
from jax import lax
from jax.experimental import pallas as pl
from jax.experimental.pallas import tpu as pltpu
from jax.experimental.pallas import tpu_sc as plsc

N = 10000
E = 320000
D = 128
DOUT = 64
EPS = 1e-5

NC = 2
NW = 32
NPT = 313
NPAD = NW * NPT
FMAX = 3.402823466e38

SCH = 3200
NSCAN = E // SCH
FLUSH = 1024
OBUF = 5120
OCAP = E + 4096

GCH = 128

_mesh = plsc.VectorSubcoreMesh(core_axis_name="c", subcore_axis_name="s")
_params = pltpu.CompilerParams(needs_layout_passes=False)


def _worker_id():
    return lax.axis_index("s") * NC + lax.axis_index("c")


@functools.partial(
    pl.kernel,
    out_type=(
        jax.ShapeDtypeStruct((NW * OCAP,), jnp.int32),
        jax.ShapeDtypeStruct((NW * OCAP,), jnp.int32),
        jax.ShapeDtypeStruct((NW * 16,), jnp.int32),
    ),
    mesh=_mesh,
    compiler_params=_params,
    scratch_types=[
        pltpu.VMEM((SCH,), jnp.int32),
        pltpu.VMEM((SCH,), jnp.int32),
        pltpu.VMEM((SCH,), jnp.int32),
        pltpu.VMEM((SCH,), jnp.int32),
        pltpu.VMEM((OBUF,), jnp.int32),
        pltpu.VMEM((OBUF,), jnp.int32),
        pltpu.VMEM((16,), jnp.int32),
        pltpu.SemaphoreType.DMA,
        pltpu.SemaphoreType.DMA,
    ],
)
def _compact(src_hbm, dst_hbm, esrc_hbm, edst_hbm, cnt_hbm,
             sbufA, dbufA, sbufB, dbufB, osrc, odst, cbuf, semA, semB):
    w = _worker_id()
    lo = w * NPT
    hi = lo + NPT

    def start_stage(ci, sb, db, sem):
        ci = jnp.minimum(ci, NSCAN - 1)
        so = pl.multiple_of(ci * SCH, 8)
        pltpu.async_copy(src_hbm.at[pl.ds(so, SCH)], sb, sem)
        pltpu.async_copy(dst_hbm.at[pl.ds(so, SCH)], db, sem)

    def wait_stage(sb, db, sem):
        pltpu.make_async_copy(src_hbm.at[pl.ds(0, SCH)], sb, sem).wait()
        pltpu.make_async_copy(dst_hbm.at[pl.ds(0, SCH)], db, sem).wait()

    def process(sb, db, carry):
        cur, base = carry

        def quad(q, cur2):
            tot = cur2
            for k in range(4):
                off = q * 64 + k * 16
                dv = db[pl.ds(off, 16)]
                sv = sb[pl.ds(off, 16)]
                m = (dv >= lo) & (dv < hi)
                cs = plsc.cumsum(m.astype(jnp.int32))
                pos = tot + cs - 1
                plsc.store_scatter(osrc, [pos], sv, mask=m)
                plsc.store_scatter(odst, [pos], dv - lo, mask=m)
                tot = tot + cs[15]
            return tot

        cur = lax.fori_loop(0, SCH // 64, quad, cur)

        nfl = cur // FLUSH
        for k in range(4):
            @pl.when(nfl >= k + 1)
            def _(k=k):
                fo = pl.multiple_of(w * OCAP + base + k * FLUSH, 8)
                pltpu.sync_copy(osrc.at[pl.ds(k * FLUSH, FLUSH)],
                                esrc_hbm.at[pl.ds(fo, FLUSH)])
                pltpu.sync_copy(odst.at[pl.ds(k * FLUSH, FLUSH)],
                                edst_hbm.at[pl.ds(fo, FLUSH)])

        sh = nfl * FLUSH

        @pl.when(nfl >= 1)
        def _():
            def mv(j, _):
                osrc[pl.ds(j * 16, 16)] = osrc[pl.ds(sh + j * 16, 16)]
                odst[pl.ds(j * 16, 16)] = odst[pl.ds(sh + j * 16, 16)]
                return 0
            lax.fori_loop(0, FLUSH // 16, mv, 0)

        return cur - sh, base + sh

    start_stage(0, sbufA, dbufA, semA)

    def pair(pi, carry):
        c0 = pi * 2
        start_stage(c0 + 1, sbufB, dbufB, semB)
        wait_stage(sbufA, dbufA, semA)
        carry = process(sbufA, dbufA, carry)
        start_stage(c0 + 2, sbufA, dbufA, semA)
        wait_stage(sbufB, dbufB, semB)
        return process(sbufB, dbufB, carry)

    cur, base = lax.fori_loop(0, NSCAN // 2, pair,
                              (jnp.int32(0), jnp.int32(0)))
    wait_stage(sbufA, dbufA, semA)

    fo = pl.multiple_of(w * OCAP + base, 8)
    pltpu.sync_copy(osrc.at[pl.ds(0, FLUSH)], esrc_hbm.at[pl.ds(fo, FLUSH)])
    pltpu.sync_copy(odst.at[pl.ds(0, FLUSH)], edst_hbm.at[pl.ds(fo, FLUSH)])
    cbuf[...] = jnp.zeros((16,), jnp.int32) + (base + cur)
    pltpu.sync_copy(cbuf, cnt_hbm.at[pl.ds(pl.multiple_of(w * 16, 8), 16)])


SUP = 16384
BCH = 256

@functools.partial(
    pl.kernel,
    out_type=jax.ShapeDtypeStruct((NW * NPT * D,), jnp.float32),
    mesh=_mesh,
    compiler_params=_params,
    scratch_types=[
        pltpu.VMEM(((NPT + 1) * D,), jnp.float32),
        pltpu.VMEM((SUP,), jnp.int32),
        pltpu.VMEM((SUP,), jnp.int32),
        pltpu.VMEM((BCH, D), jnp.float32),
        pltpu.VMEM((16,), jnp.int32),
        pltpu.SemaphoreType.DMA,
    ],
)
def _segmin(x_hbm, esrc_hbm, edst_hbm, cnt_hbm, agg_hbm,
            acc, idxs, dsts, rows, cntb, sem):
    w = _worker_id()

    fv = jnp.full((16,), FMAX, jnp.float32)

    def init(i, _):
        acc[pl.ds(i * 16, 16)] = fv
        return 0
    lax.fori_loop(0, (NPT + 1) * D // 16, init, 0)

    pltpu.sync_copy(cnt_hbm.at[pl.ds(pl.multiple_of(w * 16, 8), 16)], cntb)
    cnt = cntb[...][0]
    iota = lax.broadcasted_iota(jnp.int32, (16,), 0)

    def sup(si, _):
        so = pl.multiple_of(w * OCAP + si * SUP, 8)
        pltpu.sync_copy(esrc_hbm.at[pl.ds(so, SUP)], idxs)
        pltpu.sync_copy(edst_hbm.at[pl.ds(so, SUP)], dsts)
        rem = jnp.minimum(cnt - si * SUP, SUP)
        nb = (rem + BCH - 1) // BCH

        def batch(bi, _):
            b0 = pl.multiple_of(bi * BCH, 8)
            for g in range(BCH // 16):
                ev = si * SUP + b0 + g * 16 + iota
                valid = ev < cnt
                iv = idxs[pl.ds(b0 + g * 16, 16)]
                dv = dsts[pl.ds(b0 + g * 16, 16)]
                idxs[pl.ds(b0 + g * 16, 16)] = jnp.where(valid, iv, 0)
                dsts[pl.ds(b0 + g * 16, 16)] = jnp.where(valid, dv, NPT)
            for k in range(BCH // 128):
                pltpu.async_copy(
                    x_hbm.at[idxs.at[pl.ds(b0 + k * 128, 128)]],
                    rows.at[pl.ds(k * 128, 128), :], sem)

            def egroup(g, _):
                dvec = dsts[pl.ds(b0 + g * 16, 16)]
                for l in range(16):
                    d = dvec[l]
                    ab = d * D
                    e = g * 16 + l
                    for j in range(D // 16):
                        av = acc[pl.ds(ab + j * 16, 16)]
                        rv = rows[e, pl.ds(j * 16, 16)]
                        acc[pl.ds(ab + j * 16, 16)] = jnp.minimum(av, rv)
                return 0

            for k in range(BCH // 128):
                pltpu.make_async_copy(
                    x_hbm.at[idxs.at[pl.ds(b0 + k * 128, 128)]],
                    rows.at[pl.ds(k * 128, 128), :], sem).wait()
                lax.fori_loop(k * 8, (k + 1) * 8, egroup, 0)
            return 0

        lax.fori_loop(0, nb, batch, 0)
        return 0

    lax.fori_loop(0, (cnt + SUP - 1) // SUP, sup, 0)
    pltpu.sync_copy(acc.at[pl.ds(0, NPT * D)],
                    agg_hbm.at[pl.ds(pl.multiple_of(w * NPT * D, 8), NPT * D)])


def _dense_body(agg_ref, h_ref, wl_ref, b_ref, wr_ref, g_ref, be_ref, o_ref):
    a = agg_ref[0:N, :]
    a = jnp.where(a > 3.0e38, 0.0, a)
    z = (jnp.dot(a, wl_ref[...], preferred_element_type=jnp.float32)
         + b_ref[...]
         + jnp.dot(h_ref[...], wr_ref[...], preferred_element_type=jnp.float32))
    m = jnp.mean(z, axis=0, keepdims=True)
    v = jnp.mean((z - m) ** 2, axis=0, keepdims=True)
    zn = (z - m) * lax.rsqrt(v + EPS) * g_ref[...] + be_ref[...]
    o_ref[...] = jnp.maximum(zn, 0.0)


def _dense(agg, h, wl, b, wr, g, be):
    return pl.pallas_call(
        _dense_body,
        out_shape=jax.ShapeDtypeStruct((N, D), jnp.float32),
    )(agg, h, wl, b, wr, g, be)


def _final_body(agg_ref, h_ref, wl_ref, b_ref, wr_ref, o_ref):
    a = agg_ref[0:N, :]
    a = jnp.where(a > 3.0e38, 0.0, a)
    z = (jnp.dot(a, wl_ref[...], preferred_element_type=jnp.float32)
         + b_ref[...]
         + jnp.dot(h_ref[...], wr_ref[...], preferred_element_type=jnp.float32))
    mx = jnp.max(z, axis=1, keepdims=True)
    ez = jnp.exp(z - mx)
    ls = jnp.log(jnp.sum(ez, axis=1, keepdims=True)) + mx
    o_ref[...] = z - ls


def _final(agg, h, wl, b, wr):
    return pl.pallas_call(
        _final_body,
        out_shape=jax.ShapeDtypeStruct((N, DOUT), jnp.float32),
    )(agg, h, wl, b, wr)


def kernel(x, edge_index, W1l, b1, W1r, g1, be1,
           W2l, b2, W2r, g2, be2, W3l, b3, W3r):
    src = edge_index[0]
    dst = edge_index[1]
    esrc, edst, cnts = _compact(src, dst)

    def seg(h):
        return _segmin(h, esrc, edst, cnts).reshape(NPAD, D)

    h1 = _dense(seg(x), x, W1l, b1.reshape(1, D), W1r,
                g1.reshape(1, D), be1.reshape(1, D))
    h2 = _dense(seg(h1), h1, W2l, b2.reshape(1, D), W2r,
                g2.reshape(1, D), be2.reshape(1, D))
    return _final(seg(h2), h2, W3l, b3.reshape(1, DOUT), W3r)
